# baseline (speedup 1.0000x reference)
import jax
import jax.numpy as jnp
from jax.experimental import pallas as pl
from jax.experimental.pallas import tpu as pltpu

_B = 16384
_N = 1000
_BLK = 1024
_GRID = _B // _BLK // 2
_HALF = _GRID

_P0 = 0.6931502950629682
_P1 = 0.49990933485337247
_P2 = 0.12560248901219037
_P3 = -0.0014526603471430727
_P4 = -0.003951283348970519


def _stats(y, lab, col):
    e = jnp.exp(y)
    mask = col == lab
    rowsum = jnp.sum(e, axis=1, keepdims=True)
    s_lse = jnp.sum(jnp.log(rowsum))
    s_e2 = jnp.sum(e * e)
    sp = (((_P4 * e + _P3) * e + _P2) * e + _P1) * e + _P0
    s_sp = jnp.sum(sp)
    tv = jnp.sum(jnp.where(mask, y, 0.0), axis=1, keepdims=True)
    s_tval = jnp.sum(tv)
    s_pt = jnp.sum(jnp.exp(tv))
    return s_e2, s_pt, s_tval, s_lse, s_sp


def _pass_body(ya_ref, yb_ref, laba_ref, labb_ref, out_ref, acc_ref):
    i = pl.program_id(0)

    @pl.when(i == 0)
    def _init():
        for k in range(5):
            acc_ref[k] = 0.0

    col = jax.lax.broadcasted_iota(jnp.int32, (_BLK, _N), 1)
    sa = _stats(ya_ref[...], laba_ref[...], col)
    sb = _stats(yb_ref[...], labb_ref[...], col)
    for k in range(5):
        acc_ref[k] += sa[k] + sb[k]

    @pl.when(i == _GRID - 1)
    def _fin():
        for k in range(5):
            out_ref[k] = acc_ref[k]


def kernel(y_pred, y_true, weights):
    lab = y_true.astype(jnp.int32).reshape(_B, 1)
    sums = pl.pallas_call(
        _pass_body,
        grid=(_GRID,),
        in_specs=[
            pl.BlockSpec((_BLK, _N), lambda i: (i, i * 0)),
            pl.BlockSpec((_BLK, _N), lambda i: (i + _HALF, i * 0)),
            pl.BlockSpec((_BLK, 1), lambda i: (i, i * 0)),
            pl.BlockSpec((_BLK, 1), lambda i: (i + _HALF, i * 0)),
        ],
        out_specs=pl.BlockSpec((5,), lambda i: (i * 0,), memory_space=pltpu.SMEM),
        out_shape=jax.ShapeDtypeStruct((5,), jnp.float32),
        scratch_shapes=[pltpu.SMEM((5,), jnp.float32)],
    )(y_pred, y_pred, lab, lab)
    s_e2 = sums[0].astype(jnp.float64)
    s_pt = sums[1].astype(jnp.float64)
    s_tval = sums[2].astype(jnp.float64)
    s_lse = sums[3].astype(jnp.float64)
    s_sp = sums[4].astype(jnp.float64)

    w = jax.nn.softplus(weights)
    bn = float(_B * _N)
    mse = (s_e2 - 2.0 * s_pt + float(_B)) / bn
    ce = (s_lse - s_tval) / float(_B)
    mlsm = (s_sp - s_pt) / bn
    return w[0] * mse + w[1] * ce + w[2] * mlsm
